# baseline (speedup 1.0000x reference)
import functools

import jax
import jax.numpy as jnp
from jax import lax
from jax.experimental import pallas as pl
from jax.experimental.pallas import tpu as pltpu
from jax.experimental.pallas import tpu_sc as plsc

EPS = 1e-5
L = 16
NCHUNK = 128


def _hsum(x, lanes):
    dnums = lax.GatherDimensionNumbers(
        offset_dims=(), collapsed_slice_dims=(0,), start_index_map=(0,))
    for k in (8, 4, 2, 1):
        x = x + lax.gather(
            x, (lanes ^ k)[:, None], dimension_numbers=dnums,
            slice_sizes=(1,),
            mode=lax.GatherScatterMode.PROMISE_IN_BOUNDS)
    return x


def _rsqrt(x):
    i = lax.bitcast_convert_type(x, jnp.int32)
    i = jnp.int32(0x5F3759DF) - lax.shift_right_logical(i, 1)
    y = lax.bitcast_convert_type(i, jnp.float32)
    half = 0.5 * x
    for _ in range(2):
        y = y * (1.5 - half * y * y)
    return y


def _make_sc_kernel(B, V, D):
    info = plsc.get_sparse_core_info()
    NC, NS = info.num_cores, info.num_subcores
    NW = NC * NS
    b_per_w = B // NW
    n_chunks = b_per_w // NCHUNK
    n_vec = D // L
    DO = 2 * D

    mesh = plsc.VectorSubcoreMesh(core_axis_name="c", subcore_axis_name="s")

    @functools.partial(
        pl.kernel,
        mesh=mesh,
        out_type=jax.ShapeDtypeStruct((B, DO), jnp.float32),
        compiler_params=pltpu.CompilerParams(use_tc_tiling_on_sc=False),
        scratch_types=[
            pltpu.VMEM((n_chunks, NCHUNK), jnp.int32),
            pltpu.VMEM((b_per_w, D), jnp.float32),
            pltpu.VMEM((b_per_w, DO), jnp.float32),
            pltpu.VMEM((D,), jnp.float32),
            pltpu.VMEM((D,), jnp.float32),
            pltpu.SemaphoreType.DMA,
            pltpu.SemaphoreType.DMA,
            pltpu.SemaphoreType.DMA,
        ],
    )
    def k(idx_hbm, table_hbm, gamma_hbm, beta_hbm, out_hbm,
          idx_v, rows_v, out_v, g_v, b_v, sem_a, sem_b, sem_o):
        wid = lax.axis_index("s") * NC + lax.axis_index("c")
        base = wid * b_per_w

        pltpu.sync_copy(idx_hbm.at[pl.ds(wid * n_chunks, n_chunks)], idx_v)
        pltpu.sync_copy(gamma_hbm, g_v)
        pltpu.sync_copy(beta_hbm, b_v)

        sems = [sem_a, sem_b]

        def issue_chunk(c):
            return pltpu.async_copy(
                table_hbm.at[idx_v.at[c]],
                rows_v.at[pl.ds(c * NCHUNK, NCHUNK)], sems[c % 2])

        gathers = [issue_chunk(0), issue_chunk(1)]

        gs = [g_v[pl.ds(j * L, L)] for j in range(n_vec)]
        bs = [b_v[pl.ds(j * L, L)] for j in range(n_vec)]
        inv_d = jnp.float32(1.0 / D)
        lanes = lax.iota(jnp.int32, L)

        def ln_row(i):
            vs = [rows_v[i, pl.ds(j * L, L)] for j in range(n_vec)]
            s = vs[0]
            q = vs[0] * vs[0]
            for j in range(1, n_vec):
                s = s + vs[j]
                q = q + vs[j] * vs[j]
            mean = _hsum(s, lanes) * inv_d
            ex2 = _hsum(q, lanes) * inv_d
            var = ex2 - mean * mean
            rstd = _rsqrt(var + EPS)
            for j in range(n_vec):
                out_v[i, pl.ds(j * L, L)] = (
                    (vs[j] - mean) * rstd * gs[j] + bs[j])

        out_copies = []
        for c in range(n_chunks):
            gathers[c].wait()
            if c + 2 < n_chunks:
                gathers.append(issue_chunk(c + 2))

            def grp(g, _):
                base_row = c * NCHUNK + g * L
                for u in range(L):
                    ln_row(base_row + u)
                return 0

            lax.fori_loop(0, NCHUNK // L, grp, 0)
            out_copies.append(pltpu.async_copy(
                out_v.at[pl.ds(c * NCHUNK, NCHUNK)],
                out_hbm.at[pl.ds(base + c * NCHUNK, NCHUNK)], sem_o))
        for cp in out_copies:
            cp.wait()

    return k


def kernel(part_ids, table, gamma, beta):
    B = part_ids.shape[0]
    V, D = table.shape
    idx = part_ids.astype(jnp.int32).reshape(B // NCHUNK, NCHUNK)
    k = _make_sc_kernel(B, V, D)
    out = k(idx, table, gamma, beta)
    return out[:, None, :D]


# speedup vs baseline: 1.7834x; 1.7834x over previous
import functools

import jax
import jax.numpy as jnp
from jax import lax
from jax.experimental import pallas as pl
from jax.experimental.pallas import tpu as pltpu
from jax.experimental.pallas import tpu_sc as plsc

EPS = 1e-5
L = 16
SUB = 8


def _hsum(x, lanes):
    dnums = lax.GatherDimensionNumbers(
        offset_dims=(), collapsed_slice_dims=(0,), start_index_map=(0,))
    for k in (8, 4, 2, 1):
        x = x + lax.gather(
            x, (lanes ^ k)[:, None], dimension_numbers=dnums,
            slice_sizes=(1,),
            mode=lax.GatherScatterMode.PROMISE_IN_BOUNDS)
    return x


def _rsqrt(x):
    i = lax.bitcast_convert_type(x, jnp.int32)
    i = jnp.int32(0x5F3759DF) - lax.shift_right_logical(i, 1)
    y = lax.bitcast_convert_type(i, jnp.float32)
    half = 0.5 * x
    for _ in range(2):
        y = y * (1.5 - half * y * y)
    return y


def _make_sc_kernel(B, V, D):
    info = plsc.get_sparse_core_info()
    NC, NS = info.num_cores, info.num_subcores
    NW = NC * NS
    b_per_w = B // NW
    n_vec = D // L

    mesh = plsc.VectorSubcoreMesh(core_axis_name="c", subcore_axis_name="s")

    @functools.partial(
        pl.kernel,
        mesh=mesh,
        out_type=jax.ShapeDtypeStruct((B, D), jnp.float32),
        compiler_params=pltpu.CompilerParams(use_tc_tiling_on_sc=True),
        scratch_types=[
            pltpu.VMEM((b_per_w,), jnp.int32),
            pltpu.VMEM((b_per_w, D), jnp.float32),
            pltpu.VMEM((D,), jnp.float32),
            pltpu.VMEM((D,), jnp.float32),
            pltpu.SemaphoreType.DMA,
            pltpu.SemaphoreType.DMA,
            pltpu.SemaphoreType.DMA,
        ],
    )
    def k(idx_hbm, table_hbm, gamma_hbm, beta_hbm, out_hbm,
          idx_v, rows_v, g_v, b_v, sem_a, sem_b, sem_o):
        wid = lax.axis_index("s") * NC + lax.axis_index("c")
        base = wid * b_per_w

        pltpu.sync_copy(idx_hbm.at[wid], idx_v)
        pltpu.sync_copy(gamma_hbm, g_v)
        pltpu.sync_copy(beta_hbm, b_v)

        def issue_group(g, sem):
            v = idx_v[pl.ds(g * L, L)]
            t_vec = lax.shift_right_logical(v, SUB.bit_length() - 1)
            s_vec = lax.bitwise_and(v, SUB - 1)
            for l in range(L):
                pltpu.async_copy(
                    table_hbm.at[t_vec[l], s_vec[l]],
                    rows_v.at[g * L + l], sem)

        BS = 128
        NB = b_per_w // BS
        GPB = BS // L
        sems = [sem_a, sem_b]

        def issue_batch(c, sem):
            def grp(g, _):
                issue_group(c * GPB + g, sem)
                return 0
            lax.fori_loop(0, GPB, grp, 0)

        def drain_batch(c, sem):
            pltpu.make_async_copy(
                out_hbm.at[pl.ds(0, BS)],
                rows_v.at[pl.ds(c * BS, BS)], sem).wait()

        gs = [g_v[pl.ds(j * L, L)] for j in range(n_vec)]
        bs = [b_v[pl.ds(j * L, L)] for j in range(n_vec)]
        inv_d = jnp.float32(1.0 / D)
        lanes = lax.iota(jnp.int32, L)

        def ln_row(i):
            vs = [rows_v[i, pl.ds(j * L, L)] for j in range(n_vec)]
            s = vs[0]
            q = vs[0] * vs[0]
            for j in range(1, n_vec):
                s = s + vs[j]
                q = q + vs[j] * vs[j]
            mean = _hsum(s, lanes) * inv_d
            ex2 = _hsum(q, lanes) * inv_d
            var = ex2 - mean * mean
            rstd = _rsqrt(var + EPS)
            for j in range(n_vec):
                rows_v[i, pl.ds(j * L, L)] = (
                    (vs[j] - mean) * rstd * gs[j] + bs[j])

        def ln_batch(c):
            def grp(g, _):
                base_row = (c * GPB + g) * L
                for u in range(L):
                    ln_row(base_row + u)
                return 0
            lax.fori_loop(0, GPB, grp, 0)

        issue_batch(0, sems[0])
        out_copies = []
        for c in range(NB):
            if c + 1 < NB:
                issue_batch(c + 1, sems[(c + 1) % 2])
            drain_batch(c, sems[c % 2])
            ln_batch(c)
            out_copies.append(pltpu.async_copy(
                rows_v.at[pl.ds(c * BS, BS)],
                out_hbm.at[pl.ds(base + c * BS, BS)], sem_o))
        for cp in out_copies:
            cp.wait()

    return k


def kernel(part_ids, table, gamma, beta):
    B = part_ids.shape[0]
    V, D = table.shape
    idx = part_ids.astype(jnp.int32).reshape(32, B // 32)
    tab3 = table.reshape(V // SUB, SUB, D)
    k = _make_sc_kernel(B, V, D)
    out = k(idx, tab3, gamma, beta)
    return out[:, None, :]


# speedup vs baseline: 1.7958x; 1.0069x over previous
import functools

import jax
import jax.numpy as jnp
from jax import lax
from jax.experimental import pallas as pl
from jax.experimental.pallas import tpu as pltpu
from jax.experimental.pallas import tpu_sc as plsc

EPS = 1e-5
L = 16
SUB = 8


def _hsum(x, lanes):
    dnums = lax.GatherDimensionNumbers(
        offset_dims=(), collapsed_slice_dims=(0,), start_index_map=(0,))
    for k in (8, 4, 2, 1):
        x = x + lax.gather(
            x, (lanes ^ k)[:, None], dimension_numbers=dnums,
            slice_sizes=(1,),
            mode=lax.GatherScatterMode.PROMISE_IN_BOUNDS)
    return x


def _rsqrt(x):
    i = lax.bitcast_convert_type(x, jnp.int32)
    i = jnp.int32(0x5F3759DF) - lax.shift_right_logical(i, 1)
    y = lax.bitcast_convert_type(i, jnp.float32)
    half = 0.5 * x
    for _ in range(2):
        y = y * (1.5 - half * y * y)
    return y


def _make_sc_kernel(B, V, D):
    info = plsc.get_sparse_core_info()
    NC, NS = info.num_cores, info.num_subcores
    NW = NC * NS
    b_per_w = B // NW
    n_vec = D // L

    mesh = plsc.VectorSubcoreMesh(core_axis_name="c", subcore_axis_name="s")

    @functools.partial(
        pl.kernel,
        mesh=mesh,
        out_type=jax.ShapeDtypeStruct((B, D), jnp.float32),
        compiler_params=pltpu.CompilerParams(use_tc_tiling_on_sc=True),
        scratch_types=[
            pltpu.VMEM((b_per_w,), jnp.int32),
            pltpu.VMEM((b_per_w, D), jnp.float32),
            pltpu.VMEM((D,), jnp.float32),
            pltpu.VMEM((D,), jnp.float32),
            pltpu.SemaphoreType.DMA,
            pltpu.SemaphoreType.DMA,
            pltpu.SemaphoreType.DMA,
        ],
    )
    def k(idx_hbm, table_hbm, gamma_hbm, beta_hbm, out_hbm,
          idx_v, rows_v, g_v, b_v, sem_a, sem_b, sem_o):
        wid = lax.axis_index("s") * NC + lax.axis_index("c")
        base = wid * b_per_w

        pltpu.sync_copy(idx_hbm.at[wid], idx_v)
        g_copy = pltpu.async_copy(gamma_hbm, g_v, sem_o)
        b_copy = pltpu.async_copy(beta_hbm, b_v, sem_o)

        def issue_group(g, sem):
            v = idx_v[pl.ds(g * L, L)]
            t_vec = lax.shift_right_logical(v, SUB.bit_length() - 1)
            s_vec = lax.bitwise_and(v, SUB - 1)
            for l in range(L):
                pltpu.async_copy(
                    table_hbm.at[t_vec[l], s_vec[l]],
                    rows_v.at[g * L + l], sem)

        BS = 128
        NB = b_per_w // BS
        GPB = BS // L
        sems = [sem_a, sem_b]

        def issue_batch(c, sem):
            def grp(g, _):
                issue_group(c * GPB + g, sem)
                return 0
            lax.fori_loop(0, GPB, grp, 0)

        def drain_batch(c, sem):
            pltpu.make_async_copy(
                out_hbm.at[pl.ds(0, BS)],
                rows_v.at[pl.ds(c * BS, BS)], sem).wait()

        inv_d = jnp.float32(1.0 / D)
        lanes = lax.iota(jnp.int32, L)
        gs = []
        bs = []

        def ln_row(i):
            vs = [rows_v[i, pl.ds(j * L, L)] for j in range(n_vec)]
            s = vs[0]
            q = vs[0] * vs[0]
            for j in range(1, n_vec):
                s = s + vs[j]
                q = q + vs[j] * vs[j]
            mean = _hsum(s, lanes) * inv_d
            ex2 = _hsum(q, lanes) * inv_d
            var = ex2 - mean * mean
            rstd = _rsqrt(var + EPS)
            for j in range(n_vec):
                rows_v[i, pl.ds(j * L, L)] = (
                    (vs[j] - mean) * rstd * gs[j] + bs[j])

        def ln_batch(c):
            def grp(g, _):
                base_row = (c * GPB + g) * L
                for u in range(L):
                    ln_row(base_row + u)
                return 0
            lax.fori_loop(0, GPB, grp, 0)

        issue_batch(0, sems[0])
        g_copy.wait()
        b_copy.wait()
        gs.extend(g_v[pl.ds(j * L, L)] for j in range(n_vec))
        bs.extend(b_v[pl.ds(j * L, L)] for j in range(n_vec))
        out_copies = []
        for c in range(NB):
            if c + 1 < NB:
                issue_batch(c + 1, sems[(c + 1) % 2])
            drain_batch(c, sems[c % 2])
            ln_batch(c)
            out_copies.append(pltpu.async_copy(
                rows_v.at[pl.ds(c * BS, BS)],
                out_hbm.at[pl.ds(base + c * BS, BS)], sem_o))
        for cp in out_copies:
            cp.wait()

    return k


def kernel(part_ids, table, gamma, beta):
    B = part_ids.shape[0]
    V, D = table.shape
    idx = part_ids.astype(jnp.int32).reshape(32, B // 32)
    tab3 = table.reshape(V // SUB, SUB, D)
    k = _make_sc_kernel(B, V, D)
    out = k(idx, tab3, gamma, beta)
    return out[:, None, :]
